# baseline (speedup 1.0000x reference)
import dataclasses

import jax
import jax.numpy as jnp
from jax import lax
from jax.experimental import pallas as pl
from jax.experimental.pallas import tpu as pltpu
from jax.experimental.pallas import tpu_sc as plsc

G = 128
D = 28
N = 1048576
V = G * G * G

NC = 2
NS = 16
NW = NC * NS
L = 16

DP = 32
W = 128
PPW = N // NW
NWIN = PPW // W

RC = 512
SKEW = RC
RPW = V // NW

CORNER_OFFS = [(dx * G + dy) * G + dz
               for dx in (0, 1) for dy in (0, 1) for dz in (0, 1)]


def _repack_body(tfm_hbm, t32_hbm, f_v, out_v, sem):
    wid = lax.axis_index("s") * NC + lax.axis_index("c")
    base = wid * RPW
    i0 = lax.iota(jnp.int32, 16) * SKEW
    i1 = i0 + (D - L) * SKEW

    @pl.loop(0, RPW, step=RC)
    def _chunk(r0):
        v0 = base + r0
        copies = [pltpu.make_async_copy(tfm_hbm.at[d, pl.ds(v0, RC)],
                                        f_v.at[pl.ds(d * SKEW, RC)], sem)
                  for d in range(D)]
        for d in range(D):
            copies[d].start()
        for d in range(D):
            copies[d].wait()

        @pl.loop(0, RC, step=4)
        def _row(r):
            rv = jnp.full((L,), r, jnp.int32)
            for u in range(4):
                ru = rv + u
                out_v[r + u, pl.ds(0, L)] = plsc.load_gather(f_v, [i0 + ru])
                out_v[r + u, pl.ds(D - L, L)] = plsc.load_gather(f_v, [i1 + ru])

        pltpu.sync_copy(out_v, t32_hbm.at[pl.ds(v0, RC)])


def _main_body(pos_hbm, table_hbm, out_hbm, pos_v, idx_v, wt_v, cor_v, out_v,
               sem):
    wid = lax.axis_index("s") * NC + lax.axis_index("c")

    @pl.loop(0, NWIN)
    def _window(win):
        base = wid * PPW + win * W
        pcopies = [pltpu.make_async_copy(pos_hbm.at[pl.ds(d * N + base, W)],
                                         pos_v.at[d], sem)
                   for d in range(3)]
        for d in range(3):
            pcopies[d].start()
        for d in range(3):
            pcopies[d].wait()

        @pl.loop(0, W, step=L)
        def _grp(g):
            xs = pos_v[0, pl.ds(g, L)] * jnp.float32(G - 1)
            ys = pos_v[1, pl.ds(g, L)] * jnp.float32(G - 1)
            zs = pos_v[2, pl.ds(g, L)] * jnp.float32(G - 1)
            x0 = jnp.minimum(jnp.maximum(xs.astype(jnp.int32), 0), G - 2)
            y0 = jnp.minimum(jnp.maximum(ys.astype(jnp.int32), 0), G - 2)
            z0 = jnp.minimum(jnp.maximum(zs.astype(jnp.int32), 0), G - 2)
            fx = xs - x0.astype(jnp.float32)
            fy = ys - y0.astype(jnp.float32)
            fz = zs - z0.astype(jnp.float32)
            gx = jnp.float32(1.0) - fx
            gy = jnp.float32(1.0) - fy
            gz = jnp.float32(1.0) - fz
            flat = (x0 * G + y0) * G + z0
            wxs = (gx, fx)
            wys = (gy, fy)
            wzs = (gz, fz)
            for c in range(8):
                dx, dy, dz = (c >> 2) & 1, (c >> 1) & 1, c & 1
                idx_v[c, pl.ds(g, L)] = flat + CORNER_OFFS[c]
                wt_v[c, pl.ds(g, L)] = wxs[dx] * wys[dy] * wzs[dz]

        copies = [pltpu.make_async_copy(table_hbm.at[idx_v.at[c]],
                                        cor_v.at[c], sem)
                  for c in range(8)]
        for c in range(8):
            copies[c].start()
        for c in range(8):
            copies[c].wait()

        @pl.loop(0, W)
        def _pt(w):
            wsp = jnp.full((L,), w, jnp.int32)
            acc0 = None
            acc1 = None
            for c in range(8):
                ws = plsc.load_gather(wt_v, [jnp.full((L,), c, jnp.int32), wsp])
                r0 = cor_v[c, w, pl.ds(0, L)]
                r1 = cor_v[c, w, pl.ds(D - L, L)]
                if acc0 is None:
                    acc0 = ws * r0
                    acc1 = ws * r1
                else:
                    acc0 = acc0 + ws * r0
                    acc1 = acc1 + ws * r1
            w28 = w * D
            out_v[pl.ds(w28, L)] = acc0
            out_v[pl.ds(w28 + D - L, L)] = acc1

        pltpu.sync_copy(out_v, out_hbm.at[pl.ds(base * D, W * D)])


def _make_cp():
    cp = pltpu.CompilerParams()
    for field, val in (("needs_layout_passes", False),
                       ("use_tc_tiling_on_sc", False)):
        if field in pltpu.CompilerParams.__dataclass_fields__:
            cp = dataclasses.replace(cp, **{field: val})
    return cp


def kernel(positions, voxel_grid):
    mesh = plsc.VectorSubcoreMesh(core_axis_name="c", subcore_axis_name="s")
    cp = _make_cp()

    repack = pl.kernel(
        _repack_body,
        out_type=jax.ShapeDtypeStruct((V, DP), jnp.float32),
        mesh=mesh,
        compiler_params=cp,
        scratch_types=[
            pltpu.VMEM((D * SKEW,), jnp.float32),
            pltpu.VMEM((RC, DP), jnp.float32),
            pltpu.SemaphoreType.DMA,
        ],
    )

    main = pl.kernel(
        _main_body,
        out_type=jax.ShapeDtypeStruct((N * D,), jnp.float32),
        mesh=mesh,
        compiler_params=cp,
        scratch_types=[
            pltpu.VMEM((3, W), jnp.float32),
            pltpu.VMEM((8, W), jnp.int32),
            pltpu.VMEM((8, W), jnp.float32),
            pltpu.VMEM((8, W, DP), jnp.float32),
            pltpu.VMEM((W * D,), jnp.float32),
            pltpu.SemaphoreType.DMA,
        ],
    )

    pfm = lax.optimization_barrier(positions.T).reshape(3 * N)
    t32 = repack(voxel_grid.T)
    out1d = main(pfm, t32)
    return out1d.reshape(N, D)


# speedup vs baseline: 1.8828x; 1.8828x over previous
import dataclasses

import jax
import jax.numpy as jnp
from jax import lax
from jax.experimental import pallas as pl
from jax.experimental.pallas import tpu as pltpu
from jax.experimental.pallas import tpu_sc as plsc

G = 128
D = 28
N = 1048576
V = G * G * G
VT = V // 128
NT = N // 128

NC = 2
NS = 16
NW = NC * NS
L = 16

DP = 32
W = 128
PPW = N // NW
NWIN = PPW // W

RC = 512
RPW = V // NW

CORNER_OFFS = [(dx * G + dy) * G + dz
               for dx in (0, 1) for dy in (0, 1) for dz in (0, 1)]


def _repack_body(e_hbm, t32_hbm, f_v, out_v, sem):
    wid = lax.axis_index("s") * NC + lax.axis_index("c")
    base = wid * RPW
    lane = lax.iota(jnp.int32, 16)
    sa = lane // 8
    ra = lane % 8
    sb = (lane + (D - L)) // 8
    rb = (lane + (D - L)) % 8

    @pl.loop(0, RPW, step=RC)
    def _chunk(r0):
        v0 = base + r0
        vb0 = v0 // 128
        copies = [pltpu.make_async_copy(
            e_hbm.at[pl.ds((rt * VT + vb0) * 8, 32)], f_v.at[rt], sem)
            for rt in range(4)]
        for rt in range(4):
            copies[rt].start()
        for rt in range(4):
            copies[rt].wait()

        @pl.loop(0, RC, step=4)
        def _row(r):
            for u in range(4):
                v = r + u
                vbl = v // 128
                vl = v - vbl * 128
                rowa = ra + jnp.full((L,), vbl * 8, jnp.int32)
                rowb = rb + jnp.full((L,), vbl * 8, jnp.int32)
                lv = jnp.full((L,), vl, jnp.int32)
                out_v[v, pl.ds(0, L)] = plsc.load_gather(f_v, [sa, rowa, lv])
                out_v[v, pl.ds(D - L, L)] = plsc.load_gather(f_v, [sb, rowb, lv])

        pltpu.sync_copy(out_v, t32_hbm.at[pl.ds(v0, RC)])


def _main_body(pos_hbm, table_hbm, out_hbm, pos_v, idx_v, wt_v, cor_v, out_v,
               sem):
    wid = lax.axis_index("s") * NC + lax.axis_index("c")

    @pl.loop(0, NWIN)
    def _window(win):
        base = wid * PPW + win * W
        pltpu.sync_copy(pos_hbm.at[pl.ds((base // 128) * 8, 8)], pos_v)

        @pl.loop(0, W, step=L)
        def _grp(g):
            xs = pos_v[0, pl.ds(g, L)] * jnp.float32(G - 1)
            ys = pos_v[1, pl.ds(g, L)] * jnp.float32(G - 1)
            zs = pos_v[2, pl.ds(g, L)] * jnp.float32(G - 1)
            x0 = jnp.minimum(jnp.maximum(xs.astype(jnp.int32), 0), G - 2)
            y0 = jnp.minimum(jnp.maximum(ys.astype(jnp.int32), 0), G - 2)
            z0 = jnp.minimum(jnp.maximum(zs.astype(jnp.int32), 0), G - 2)
            fx = xs - x0.astype(jnp.float32)
            fy = ys - y0.astype(jnp.float32)
            fz = zs - z0.astype(jnp.float32)
            gx = jnp.float32(1.0) - fx
            gy = jnp.float32(1.0) - fy
            gz = jnp.float32(1.0) - fz
            flat = (x0 * G + y0) * G + z0
            wxs = (gx, fx)
            wys = (gy, fy)
            wzs = (gz, fz)
            for c in range(8):
                dx, dy, dz = (c >> 2) & 1, (c >> 1) & 1, c & 1
                idx_v[c, pl.ds(g, L)] = flat + CORNER_OFFS[c]
                wt_v[c, pl.ds(g, L)] = wxs[dx] * wys[dy] * wzs[dz]

        copies = [pltpu.make_async_copy(table_hbm.at[idx_v.at[c]],
                                        cor_v.at[c], sem)
                  for c in range(8)]
        for c in range(8):
            copies[c].start()
        for c in range(8):
            copies[c].wait()

        @pl.loop(0, W)
        def _pt(w):
            wsp = jnp.full((L,), w, jnp.int32)
            acc0 = None
            acc1 = None
            for c in range(8):
                ws = plsc.load_gather(wt_v, [jnp.full((L,), c, jnp.int32), wsp])
                r0 = cor_v[c, w, pl.ds(0, L)]
                r1 = cor_v[c, w, pl.ds(D - L, L)]
                if acc0 is None:
                    acc0 = ws * r0
                    acc1 = ws * r1
                else:
                    acc0 = acc0 + ws * r0
                    acc1 = acc1 + ws * r1
            w28 = w * D
            out_v[pl.ds(w28, L)] = acc0
            out_v[pl.ds(w28 + D - L, L)] = acc1

        pltpu.sync_copy(out_v, out_hbm.at[pl.ds(base * D, W * D)])


def _make_cp():
    cp = pltpu.CompilerParams()
    for field, val in (("needs_layout_passes", False),
                       ("use_tc_tiling_on_sc", False)):
        if field in pltpu.CompilerParams.__dataclass_fields__:
            cp = dataclasses.replace(cp, **{field: val})
    return cp


def kernel(positions, voxel_grid):
    mesh = plsc.VectorSubcoreMesh(core_axis_name="c", subcore_axis_name="s")
    cp = _make_cp()

    repack = pl.kernel(
        _repack_body,
        out_type=jax.ShapeDtypeStruct((V, DP), jnp.float32),
        mesh=mesh,
        compiler_params=cp,
        scratch_types=[
            pltpu.VMEM((4, 32, 128), jnp.float32),
            pltpu.VMEM((RC, DP), jnp.float32),
            pltpu.SemaphoreType.DMA,
        ],
    )

    main = pl.kernel(
        _main_body,
        out_type=jax.ShapeDtypeStruct((N * D,), jnp.float32),
        mesh=mesh,
        compiler_params=cp,
        scratch_types=[
            pltpu.VMEM((8, W), jnp.float32),
            pltpu.VMEM((8, W), jnp.int32),
            pltpu.VMEM((8, W), jnp.float32),
            pltpu.VMEM((8, W, DP), jnp.float32),
            pltpu.VMEM((W * D,), jnp.float32),
            pltpu.SemaphoreType.DMA,
        ],
    )

    t_e = (jnp.pad(voxel_grid.T, ((0, 4), (0, 0)))
           .reshape(4, 8, VT, 128).transpose(0, 2, 1, 3).reshape(-1, 128))
    p_e = (jnp.pad(positions.T, ((0, 5), (0, 0)))
           .reshape(1, 8, NT, 128).transpose(0, 2, 1, 3).reshape(-1, 128))

    t32 = repack(t_e)
    out1d = main(p_e, t32)
    return out1d.reshape(N, D)


# speedup vs baseline: 2.0092x; 1.0671x over previous
import dataclasses

import jax
import jax.numpy as jnp
from jax import lax
from jax.experimental import pallas as pl
from jax.experimental.pallas import tpu as pltpu
from jax.experimental.pallas import tpu_sc as plsc

G = 128
D = 28
N = 1048576
V = G * G * G
VT = V // 128
NT = N // 128

NC = 2
NS = 16
NW = NC * NS
L = 16

DP = 32
W = 128
PPW = N // NW
NWIN = PPW // W

RC = 512
RPW = V // NW

CORNER_OFFS = [(dx * G + dy) * G + dz
               for dx in (0, 1) for dy in (0, 1) for dz in (0, 1)]


def _repack_body(e_hbm, t32_hbm, f_v, out_v, sem_in, sem_out):
    wid = lax.axis_index("s") * NC + lax.axis_index("c")
    base = wid * RPW
    nchunk = RPW // RC
    lane = lax.iota(jnp.int32, 16)
    sa = lane // 8
    ra = lane % 8
    sb = (lane + (D - L)) // 8
    rb = (lane + (D - L)) % 8

    def in_copies(ci, b):
        vb0 = (base + ci * RC) // 128
        return [pltpu.make_async_copy(
            e_hbm.at[pl.ds((rt * VT + vb0) * 8, 32)], f_v.at[b, rt], sem_in)
            for rt in range(4)]

    def out_copy(ci, b):
        return pltpu.make_async_copy(
            out_v.at[b], t32_hbm.at[pl.ds(base + ci * RC, RC)], sem_out)

    for b in range(2):
        for cp in in_copies(b, b):
            cp.start()

    @pl.loop(0, nchunk, step=2)
    def _chunk(c):
        for b in range(2):
            ci = c + b
            for cp in in_copies(ci, b):
                cp.wait()

            @pl.when(ci >= 2)
            def _():
                out_copy(ci - 2, b).wait()

            @pl.loop(0, RC, step=4)
            def _row(r):
                for u in range(4):
                    v = r + u
                    vbl = v // 128
                    vl = v - vbl * 128
                    rowa = ra + jnp.full((L,), vbl * 8, jnp.int32)
                    rowb = rb + jnp.full((L,), vbl * 8, jnp.int32)
                    lv = jnp.full((L,), vl, jnp.int32)
                    out_v[b, v, pl.ds(0, L)] = plsc.load_gather(
                        f_v.at[b], [sa, rowa, lv])
                    out_v[b, v, pl.ds(D - L, L)] = plsc.load_gather(
                        f_v.at[b], [sb, rowb, lv])

            out_copy(ci, b).start()

            @pl.when(ci + 2 < nchunk)
            def _():
                for cp in in_copies(ci + 2, b):
                    cp.start()

    for b in range(2):
        out_copy(nchunk - 2 + b, b).wait()


def _main_body(pos_hbm, table_hbm, out_hbm, pos_v, idx_v, wt_v, cor_v, out_v,
               sem):
    wid = lax.axis_index("s") * NC + lax.axis_index("c")

    @pl.loop(0, NWIN)
    def _window(win):
        base = wid * PPW + win * W
        pltpu.sync_copy(pos_hbm.at[pl.ds((base // 128) * 8, 8)], pos_v)

        @pl.loop(0, W, step=L)
        def _grp(g):
            xs = pos_v[0, pl.ds(g, L)] * jnp.float32(G - 1)
            ys = pos_v[1, pl.ds(g, L)] * jnp.float32(G - 1)
            zs = pos_v[2, pl.ds(g, L)] * jnp.float32(G - 1)
            x0 = jnp.minimum(jnp.maximum(xs.astype(jnp.int32), 0), G - 2)
            y0 = jnp.minimum(jnp.maximum(ys.astype(jnp.int32), 0), G - 2)
            z0 = jnp.minimum(jnp.maximum(zs.astype(jnp.int32), 0), G - 2)
            fx = xs - x0.astype(jnp.float32)
            fy = ys - y0.astype(jnp.float32)
            fz = zs - z0.astype(jnp.float32)
            gx = jnp.float32(1.0) - fx
            gy = jnp.float32(1.0) - fy
            gz = jnp.float32(1.0) - fz
            flat = (x0 * G + y0) * G + z0
            wxs = (gx, fx)
            wys = (gy, fy)
            wzs = (gz, fz)
            for c in range(8):
                dx, dy, dz = (c >> 2) & 1, (c >> 1) & 1, c & 1
                idx_v[c, pl.ds(g, L)] = flat + CORNER_OFFS[c]
                wt_v[c, pl.ds(g, L)] = wxs[dx] * wys[dy] * wzs[dz]

        copies = [pltpu.make_async_copy(table_hbm.at[idx_v.at[c]],
                                        cor_v.at[c], sem)
                  for c in range(8)]
        for c in range(8):
            copies[c].start()
        for c in range(8):
            copies[c].wait()

        @pl.loop(0, W)
        def _pt(w):
            wsp = jnp.full((L,), w, jnp.int32)
            acc0 = None
            acc1 = None
            for c in range(8):
                ws = plsc.load_gather(wt_v, [jnp.full((L,), c, jnp.int32), wsp])
                r0 = cor_v[c, w, pl.ds(0, L)]
                r1 = cor_v[c, w, pl.ds(D - L, L)]
                if acc0 is None:
                    acc0 = ws * r0
                    acc1 = ws * r1
                else:
                    acc0 = acc0 + ws * r0
                    acc1 = acc1 + ws * r1
            w28 = w * D
            out_v[pl.ds(w28, L)] = acc0
            out_v[pl.ds(w28 + D - L, L)] = acc1

        pltpu.sync_copy(out_v, out_hbm.at[pl.ds(base * D, W * D)])


def _make_cp():
    cp = pltpu.CompilerParams()
    for field, val in (("needs_layout_passes", False),
                       ("use_tc_tiling_on_sc", False)):
        if field in pltpu.CompilerParams.__dataclass_fields__:
            cp = dataclasses.replace(cp, **{field: val})
    return cp


def kernel(positions, voxel_grid):
    mesh = plsc.VectorSubcoreMesh(core_axis_name="c", subcore_axis_name="s")
    cp = _make_cp()

    repack = pl.kernel(
        _repack_body,
        out_type=jax.ShapeDtypeStruct((V, DP), jnp.float32),
        mesh=mesh,
        compiler_params=cp,
        scratch_types=[
            pltpu.VMEM((2, 4, 32, 128), jnp.float32),
            pltpu.VMEM((2, RC, DP), jnp.float32),
            pltpu.SemaphoreType.DMA,
            pltpu.SemaphoreType.DMA,
        ],
    )

    main = pl.kernel(
        _main_body,
        out_type=jax.ShapeDtypeStruct((N * D,), jnp.float32),
        mesh=mesh,
        compiler_params=cp,
        scratch_types=[
            pltpu.VMEM((8, W), jnp.float32),
            pltpu.VMEM((8, W), jnp.int32),
            pltpu.VMEM((8, W), jnp.float32),
            pltpu.VMEM((8, W, DP), jnp.float32),
            pltpu.VMEM((W * D,), jnp.float32),
            pltpu.SemaphoreType.DMA,
        ],
    )

    t_e = (jnp.pad(voxel_grid.T, ((0, 4), (0, 0)))
           .reshape(4, 8, VT, 128).transpose(0, 2, 1, 3).reshape(-1, 128))
    p_e = (jnp.pad(positions.T, ((0, 5), (0, 0)))
           .reshape(1, 8, NT, 128).transpose(0, 2, 1, 3).reshape(-1, 128))

    t32 = repack(t_e)
    out1d = main(p_e, t32)
    return out1d.reshape(N, D)


# speedup vs baseline: 2.4360x; 1.2124x over previous
import dataclasses

import jax
import jax.numpy as jnp
from jax import lax
from jax.experimental import pallas as pl
from jax.experimental.pallas import tpu as pltpu
from jax.experimental.pallas import tpu_sc as plsc

G = 128
D = 28
N = 1048576
V = G * G * G
VT = V // 128
NT = N // 128

NC = 2
NS = 16
NW = NC * NS
L = 16

DP = 32
W = 128
PPW = N // NW
NWIN = PPW // W

RC = 512
RPW = V // NW

CORNER_OFFS = [(dx * G + dy) * G + dz
               for dx in (0, 1) for dy in (0, 1) for dz in (0, 1)]


def _repack_body(e_hbm, t32_hbm, f_v, out_v, sem_in, sem_out):
    wid = lax.axis_index("s") * NC + lax.axis_index("c")
    base = wid * RPW
    nchunk = RPW // RC
    lane = lax.iota(jnp.int32, 16)
    sa = lane // 8
    ra = lane % 8
    sb = (lane + (D - L)) // 8
    rb = (lane + (D - L)) % 8

    def in_copies(ci, b):
        vb0 = (base + ci * RC) // 128
        return [pltpu.make_async_copy(
            e_hbm.at[pl.ds((rt * VT + vb0) * 8, 32)], f_v.at[b, rt], sem_in)
            for rt in range(4)]

    def out_copy(ci, b):
        return pltpu.make_async_copy(
            out_v.at[b], t32_hbm.at[pl.ds(base + ci * RC, RC)], sem_out)

    for b in range(2):
        for cp in in_copies(b, b):
            cp.start()

    @pl.loop(0, nchunk, step=2)
    def _chunk(c):
        for b in range(2):
            ci = c + b
            for cp in in_copies(ci, b):
                cp.wait()

            @pl.when(ci >= 2)
            def _():
                out_copy(ci - 2, b).wait()

            @pl.loop(0, RC, step=4)
            def _row(r):
                for u in range(4):
                    v = r + u
                    vbl = v // 128
                    vl = v - vbl * 128
                    rowa = ra + jnp.full((L,), vbl * 8, jnp.int32)
                    rowb = rb + jnp.full((L,), vbl * 8, jnp.int32)
                    lv = jnp.full((L,), vl, jnp.int32)
                    out_v[b, v, pl.ds(0, L)] = plsc.load_gather(
                        f_v.at[b], [sa, rowa, lv])
                    out_v[b, v, pl.ds(D - L, L)] = plsc.load_gather(
                        f_v.at[b], [sb, rowb, lv])

            out_copy(ci, b).start()

            @pl.when(ci + 2 < nchunk)
            def _():
                for cp in in_copies(ci + 2, b):
                    cp.start()

    for b in range(2):
        out_copy(nchunk - 2 + b, b).wait()


def _main_body(pos_hbm, table_hbm, out_hbm, pos_v, idx_v, wt_v, cor_v, out_v,
               sem_p, sem_g, sem_o):
    wid = lax.axis_index("s") * NC + lax.axis_index("c")
    wbase = wid * PPW

    def pos_copy(ki, b):
        return pltpu.make_async_copy(
            pos_hbm.at[pl.ds(((wbase + ki * W) // 128) * 8, 8)],
            pos_v.at[b], sem_p)

    def gather_copies(b):
        return [pltpu.make_async_copy(table_hbm.at[idx_v.at[b, c]],
                                      cor_v.at[b, c], sem_g)
                for c in range(8)]

    def out_copy(ki, b):
        return pltpu.make_async_copy(
            out_v.at[b], out_hbm.at[pl.ds((wbase + ki * W) * D, W * D)],
            sem_o)

    def compute_idx_wt(b):
        @pl.loop(0, W, step=L)
        def _grp(g):
            xs = pos_v[b, 0, pl.ds(g, L)] * jnp.float32(G - 1)
            ys = pos_v[b, 1, pl.ds(g, L)] * jnp.float32(G - 1)
            zs = pos_v[b, 2, pl.ds(g, L)] * jnp.float32(G - 1)
            x0 = jnp.minimum(jnp.maximum(xs.astype(jnp.int32), 0), G - 2)
            y0 = jnp.minimum(jnp.maximum(ys.astype(jnp.int32), 0), G - 2)
            z0 = jnp.minimum(jnp.maximum(zs.astype(jnp.int32), 0), G - 2)
            fx = xs - x0.astype(jnp.float32)
            fy = ys - y0.astype(jnp.float32)
            fz = zs - z0.astype(jnp.float32)
            gx = jnp.float32(1.0) - fx
            gy = jnp.float32(1.0) - fy
            gz = jnp.float32(1.0) - fz
            flat = (x0 * G + y0) * G + z0
            wxs = (gx, fx)
            wys = (gy, fy)
            wzs = (gz, fz)
            for c in range(8):
                dx, dy, dz = (c >> 2) & 1, (c >> 1) & 1, c & 1
                idx_v[b, c, pl.ds(g, L)] = flat + CORNER_OFFS[c]
                wt_v[b, c, pl.ds(g, L)] = wxs[dx] * wys[dy] * wzs[dz]

    def blend(b):
        @pl.loop(0, W)
        def _pt(w):
            wsp = jnp.full((L,), w, jnp.int32)
            acc0 = None
            acc1 = None
            for c in range(8):
                ws = plsc.load_gather(
                    wt_v.at[b], [jnp.full((L,), c, jnp.int32), wsp])
                r0 = cor_v[b, c, w, pl.ds(0, L)]
                r1 = cor_v[b, c, w, pl.ds(D - L, L)]
                if acc0 is None:
                    acc0 = ws * r0
                    acc1 = ws * r1
                else:
                    acc0 = acc0 + ws * r0
                    acc1 = acc1 + ws * r1
            w28 = w * D
            out_v[b, pl.ds(w28, L)] = acc0
            out_v[b, pl.ds(w28 + D - L, L)] = acc1

    pos_copy(0, 0).start()

    @pl.loop(0, NWIN, step=2)
    def _window(c):
        for b in range(2):
            ki = c + b
            nb = 1 - b
            pos_copy(ki, b).wait()

            @pl.when(ki + 1 < NWIN)
            def _():
                pos_copy(ki + 1, nb).start()

            compute_idx_wt(b)
            for cp in gather_copies(b):
                cp.start()

            @pl.when(ki >= 1)
            def _():
                @pl.when(ki >= 3)
                def _():
                    out_copy(ki - 3, nb).wait()
                for cp in gather_copies(nb):
                    cp.wait()
                blend(nb)
                out_copy(ki - 1, nb).start()

    out_copy(NWIN - 3, 1).wait()
    for cp in gather_copies(1):
        cp.wait()
    blend(1)
    out_copy(NWIN - 1, 1).start()
    out_copy(NWIN - 2, 0).wait()
    out_copy(NWIN - 1, 1).wait()


def _make_cp():
    cp = pltpu.CompilerParams()
    for field, val in (("needs_layout_passes", False),
                       ("use_tc_tiling_on_sc", False)):
        if field in pltpu.CompilerParams.__dataclass_fields__:
            cp = dataclasses.replace(cp, **{field: val})
    return cp


def kernel(positions, voxel_grid):
    mesh = plsc.VectorSubcoreMesh(core_axis_name="c", subcore_axis_name="s")
    cp = _make_cp()

    repack = pl.kernel(
        _repack_body,
        out_type=jax.ShapeDtypeStruct((V, DP), jnp.float32),
        mesh=mesh,
        compiler_params=cp,
        scratch_types=[
            pltpu.VMEM((2, 4, 32, 128), jnp.float32),
            pltpu.VMEM((2, RC, DP), jnp.float32),
            pltpu.SemaphoreType.DMA,
            pltpu.SemaphoreType.DMA,
        ],
    )

    main = pl.kernel(
        _main_body,
        out_type=jax.ShapeDtypeStruct((N * D,), jnp.float32),
        mesh=mesh,
        compiler_params=cp,
        scratch_types=[
            pltpu.VMEM((2, 8, W), jnp.float32),
            pltpu.VMEM((2, 8, W), jnp.int32),
            pltpu.VMEM((2, 8, W), jnp.float32),
            pltpu.VMEM((2, 8, W, DP), jnp.float32),
            pltpu.VMEM((2, W * D), jnp.float32),
            pltpu.SemaphoreType.DMA,
            pltpu.SemaphoreType.DMA,
            pltpu.SemaphoreType.DMA,
        ],
    )

    t_e = (jnp.pad(voxel_grid.T, ((0, 4), (0, 0)))
           .reshape(4, 8, VT, 128).transpose(0, 2, 1, 3).reshape(-1, 128))
    p_e = (jnp.pad(positions.T, ((0, 5), (0, 0)))
           .reshape(1, 8, NT, 128).transpose(0, 2, 1, 3).reshape(-1, 128))

    t32 = repack(t_e)
    out1d = main(p_e, t32)
    return out1d.reshape(N, D)


# speedup vs baseline: 2.8317x; 1.1624x over previous
import dataclasses

import jax
import jax.numpy as jnp
from jax import lax
from jax.experimental import pallas as pl
from jax.experimental.pallas import tpu as pltpu
from jax.experimental.pallas import tpu_sc as plsc

G = 128
D = 28
N = 1048576
V = G * G * G
VT = V // 128
NT = N // 128

NC = 2
NS = 16
NW = NC * NS
L = 16

DP = 32
W = 128
PPW = N // NW
NWIN = PPW // W

RC = 512
RPW = V // NW

CORNER_OFFS = [(dx * G + dy) * G + dz
               for dx in (0, 1) for dy in (0, 1) for dz in (0, 1)]


def _repack_body(e_hbm, t32_hbm, f_v, out_v, sem_in, sem_out):
    wid = lax.axis_index("s") * NC + lax.axis_index("c")
    base = wid * RPW
    nchunk = RPW // RC
    lane = lax.iota(jnp.int32, 16)
    cds = [jnp.full((L,), d, jnp.int32) for d in range(D)]

    def in_copies(ci, b):
        vb0 = (base + ci * RC) // 128
        return [pltpu.make_async_copy(
            e_hbm.at[pl.ds((rt * VT + vb0) * 8, 32)], f_v.at[b, rt], sem_in)
            for rt in range(4)]

    def out_copy(ci, b):
        return pltpu.make_async_copy(
            out_v.at[b], t32_hbm.at[pl.ds(base + ci * RC, RC)], sem_out)

    for b in range(2):
        for cp in in_copies(b, b):
            cp.start()

    @pl.loop(0, nchunk, step=2)
    def _chunk(c):
        for b in range(2):
            ci = c + b
            for cp in in_copies(ci, b):
                cp.wait()

            @pl.when(ci >= 2)
            def _():
                out_copy(ci - 2, b).wait()

            @pl.loop(0, RC, step=L)
            def _grp(g):
                vbl = g // 128
                vl0 = g - vbl * 128
                row0 = vbl * 8
                rows = lane + g
                for d in range(D):
                    val = f_v[b, d // 8, row0 + (d % 8), pl.ds(vl0, L)]
                    plsc.store_scatter(out_v.at[b], [rows, cds[d]], val)

            out_copy(ci, b).start()

            @pl.when(ci + 2 < nchunk)
            def _():
                for cp in in_copies(ci + 2, b):
                    cp.start()

    for b in range(2):
        out_copy(nchunk - 2 + b, b).wait()


def _main_body(pos_hbm, table_hbm, out_hbm, pos_v, idx_v, wt_v, cor_v, out_v,
               sem_p, sem_g, sem_o):
    wid = lax.axis_index("s") * NC + lax.axis_index("c")
    wbase = wid * PPW

    def pos_copy(ki, b):
        return pltpu.make_async_copy(
            pos_hbm.at[pl.ds(((wbase + ki * W) // 128) * 8, 8)],
            pos_v.at[b], sem_p)

    def gather_copies(b):
        return [pltpu.make_async_copy(table_hbm.at[idx_v.at[b, c]],
                                      cor_v.at[b, c], sem_g)
                for c in range(8)]

    def out_copy(ki, b):
        return pltpu.make_async_copy(
            out_v.at[b], out_hbm.at[pl.ds((wbase + ki * W) * D, W * D)],
            sem_o)

    def compute_idx_wt(b):
        @pl.loop(0, W, step=L)
        def _grp(g):
            xs = pos_v[b, 0, pl.ds(g, L)] * jnp.float32(G - 1)
            ys = pos_v[b, 1, pl.ds(g, L)] * jnp.float32(G - 1)
            zs = pos_v[b, 2, pl.ds(g, L)] * jnp.float32(G - 1)
            x0 = jnp.minimum(jnp.maximum(xs.astype(jnp.int32), 0), G - 2)
            y0 = jnp.minimum(jnp.maximum(ys.astype(jnp.int32), 0), G - 2)
            z0 = jnp.minimum(jnp.maximum(zs.astype(jnp.int32), 0), G - 2)
            fx = xs - x0.astype(jnp.float32)
            fy = ys - y0.astype(jnp.float32)
            fz = zs - z0.astype(jnp.float32)
            gx = jnp.float32(1.0) - fx
            gy = jnp.float32(1.0) - fy
            gz = jnp.float32(1.0) - fz
            flat = (x0 * G + y0) * G + z0
            wxs = (gx, fx)
            wys = (gy, fy)
            wzs = (gz, fz)
            for c in range(8):
                dx, dy, dz = (c >> 2) & 1, (c >> 1) & 1, c & 1
                idx_v[b, c, pl.ds(g, L)] = flat + CORNER_OFFS[c]
                wt_v[b, c, pl.ds(g, L)] = wxs[dx] * wys[dy] * wzs[dz]

    def blend(b):
        @pl.loop(0, W)
        def _pt(w):
            wsp = jnp.full((L,), w, jnp.int32)
            acc0 = None
            acc1 = None
            for c in range(8):
                ws = plsc.load_gather(
                    wt_v.at[b], [jnp.full((L,), c, jnp.int32), wsp])
                r0 = cor_v[b, c, w, pl.ds(0, L)]
                r1 = cor_v[b, c, w, pl.ds(D - L, L)]
                if acc0 is None:
                    acc0 = ws * r0
                    acc1 = ws * r1
                else:
                    acc0 = acc0 + ws * r0
                    acc1 = acc1 + ws * r1
            w28 = w * D
            out_v[b, pl.ds(w28, L)] = acc0
            out_v[b, pl.ds(w28 + D - L, L)] = acc1

    pos_copy(0, 0).start()

    @pl.loop(0, NWIN, step=2)
    def _window(c):
        for b in range(2):
            ki = c + b
            nb = 1 - b
            pos_copy(ki, b).wait()

            @pl.when(ki + 1 < NWIN)
            def _():
                pos_copy(ki + 1, nb).start()

            compute_idx_wt(b)
            for cp in gather_copies(b):
                cp.start()

            @pl.when(ki >= 1)
            def _():
                @pl.when(ki >= 3)
                def _():
                    out_copy(ki - 3, nb).wait()
                for cp in gather_copies(nb):
                    cp.wait()
                blend(nb)
                out_copy(ki - 1, nb).start()

    out_copy(NWIN - 3, 1).wait()
    for cp in gather_copies(1):
        cp.wait()
    blend(1)
    out_copy(NWIN - 1, 1).start()
    out_copy(NWIN - 2, 0).wait()
    out_copy(NWIN - 1, 1).wait()


def _make_cp():
    cp = pltpu.CompilerParams()
    for field, val in (("needs_layout_passes", False),
                       ("use_tc_tiling_on_sc", False)):
        if field in pltpu.CompilerParams.__dataclass_fields__:
            cp = dataclasses.replace(cp, **{field: val})
    return cp


def kernel(positions, voxel_grid):
    mesh = plsc.VectorSubcoreMesh(core_axis_name="c", subcore_axis_name="s")
    cp = _make_cp()

    repack = pl.kernel(
        _repack_body,
        out_type=jax.ShapeDtypeStruct((V, DP), jnp.float32),
        mesh=mesh,
        compiler_params=cp,
        scratch_types=[
            pltpu.VMEM((2, 4, 32, 128), jnp.float32),
            pltpu.VMEM((2, RC, DP), jnp.float32),
            pltpu.SemaphoreType.DMA,
            pltpu.SemaphoreType.DMA,
        ],
    )

    main = pl.kernel(
        _main_body,
        out_type=jax.ShapeDtypeStruct((N * D,), jnp.float32),
        mesh=mesh,
        compiler_params=cp,
        scratch_types=[
            pltpu.VMEM((2, 8, W), jnp.float32),
            pltpu.VMEM((2, 8, W), jnp.int32),
            pltpu.VMEM((2, 8, W), jnp.float32),
            pltpu.VMEM((2, 8, W, DP), jnp.float32),
            pltpu.VMEM((2, W * D), jnp.float32),
            pltpu.SemaphoreType.DMA,
            pltpu.SemaphoreType.DMA,
            pltpu.SemaphoreType.DMA,
        ],
    )

    t_e = (jnp.pad(voxel_grid.T, ((0, 4), (0, 0)))
           .reshape(4, 8, VT, 128).transpose(0, 2, 1, 3).reshape(-1, 128))
    p_e = (jnp.pad(positions.T, ((0, 5), (0, 0)))
           .reshape(1, 8, NT, 128).transpose(0, 2, 1, 3).reshape(-1, 128))

    t32 = repack(t_e)
    out1d = main(p_e, t32)
    return out1d.reshape(N, D)


# speedup vs baseline: 3.1494x; 1.1122x over previous
import dataclasses

import jax
import jax.numpy as jnp
from jax import lax
from jax.experimental import pallas as pl
from jax.experimental.pallas import tpu as pltpu
from jax.experimental.pallas import tpu_sc as plsc

G = 128
D = 28
N = 1048576
V = G * G * G
VT = V // 128
NT = N // 128

NC = 2
NS = 16
NW = NC * NS
L = 16

DP = 32
W = 128
PPW = N // NW
NWIN = PPW // W

RC = 512
RPW = V // NW

CORNER_OFFS = [(dx * G + dy) * G + dz
               for dx in (0, 1) for dy in (0, 1) for dz in (0, 1)]


def _repack_body(e_hbm, t32_hbm, f_v, out_v, sem_in, sem_out):
    wid = lax.axis_index("s") * NC + lax.axis_index("c")
    base = wid * RPW
    nchunk = RPW // RC
    lane = lax.iota(jnp.int32, 16)
    cds = [jnp.full((L,), d, jnp.int32) for d in range(D)]

    def in_copies(ci, b):
        vb0 = (base + ci * RC) // 128
        return [pltpu.make_async_copy(
            e_hbm.at[pl.ds((rt * VT + vb0) * 8, 32)], f_v.at[b, rt], sem_in)
            for rt in range(4)]

    def out_copy(ci, b):
        return pltpu.make_async_copy(
            out_v.at[b], t32_hbm.at[pl.ds(base + ci * RC, RC)], sem_out)

    for b in range(2):
        for cp in in_copies(b, b):
            cp.start()

    @pl.loop(0, nchunk, step=2)
    def _chunk(c):
        for b in range(2):
            ci = c + b
            for cp in in_copies(ci, b):
                cp.wait()

            @pl.when(ci >= 2)
            def _():
                out_copy(ci - 2, b).wait()

            @pl.loop(0, RC, step=L)
            def _grp(g):
                vbl = g // 128
                vl0 = g - vbl * 128
                row0 = vbl * 8
                rows = lane + g
                for d in range(D):
                    val = f_v[b, d // 8, row0 + (d % 8), pl.ds(vl0, L)]
                    plsc.store_scatter(out_v.at[b], [rows, cds[d]], val)

            out_copy(ci, b).start()

            @pl.when(ci + 2 < nchunk)
            def _():
                for cp in in_copies(ci + 2, b):
                    cp.start()

    for b in range(2):
        out_copy(nchunk - 2 + b, b).wait()


def _main_body(pos_hbm, table_hbm, out_hbm, pos_v, idx_v, wt_v, cor_v, out_v,
               sem_p, sem_g, sem_o):
    wid = lax.axis_index("s") * NC + lax.axis_index("c")
    wbase = wid * PPW
    lane = lax.iota(jnp.int32, 16)
    rows_b = lane + (D - L)

    def pos_copy(ki, b):
        return pltpu.make_async_copy(
            pos_hbm.at[pl.ds(((wbase + ki * W) // 128) * 8, 8)],
            pos_v.at[b], sem_p)

    def gather_copies(b):
        return [pltpu.make_async_copy(table_hbm.at[idx_v.at[b, c]],
                                      cor_v.at[b, c], sem_g)
                for c in range(8)]

    def out_copies(ki, b):
        vb = (wbase + ki * W) // 128
        return [pltpu.make_async_copy(
            out_v.at[b, pl.ds(rt * 8, 8)],
            out_hbm.at[pl.ds((rt * NT + vb) * 8, 8)], sem_o)
            for rt in range(4)]

    def compute_idx_wt(b):
        @pl.loop(0, W, step=L)
        def _grp(g):
            xs = pos_v[b, 0, pl.ds(g, L)] * jnp.float32(G - 1)
            ys = pos_v[b, 1, pl.ds(g, L)] * jnp.float32(G - 1)
            zs = pos_v[b, 2, pl.ds(g, L)] * jnp.float32(G - 1)
            x0 = jnp.minimum(jnp.maximum(xs.astype(jnp.int32), 0), G - 2)
            y0 = jnp.minimum(jnp.maximum(ys.astype(jnp.int32), 0), G - 2)
            z0 = jnp.minimum(jnp.maximum(zs.astype(jnp.int32), 0), G - 2)
            fx = xs - x0.astype(jnp.float32)
            fy = ys - y0.astype(jnp.float32)
            fz = zs - z0.astype(jnp.float32)
            gx = jnp.float32(1.0) - fx
            gy = jnp.float32(1.0) - fy
            gz = jnp.float32(1.0) - fz
            flat = (x0 * G + y0) * G + z0
            wxs = (gx, fx)
            wys = (gy, fy)
            wzs = (gz, fz)
            for c in range(8):
                dx, dy, dz = (c >> 2) & 1, (c >> 1) & 1, c & 1
                idx_v[b, c, pl.ds(g, L)] = flat + CORNER_OFFS[c]
                wt_v[b, c, pl.ds(g, L)] = wxs[dx] * wys[dy] * wzs[dz]

    def blend(b):
        @pl.loop(0, W)
        def _pt(w):
            wsp = jnp.full((L,), w, jnp.int32)
            acc0 = None
            acc1 = None
            for c in range(8):
                ws = plsc.load_gather(
                    wt_v.at[b], [jnp.full((L,), c, jnp.int32), wsp])
                r0 = cor_v[b, c, w, pl.ds(0, L)]
                r1 = cor_v[b, c, w, pl.ds(D - L, L)]
                if acc0 is None:
                    acc0 = ws * r0
                    acc1 = ws * r1
                else:
                    acc0 = acc0 + ws * r0
                    acc1 = acc1 + ws * r1
            plsc.store_scatter(out_v.at[b], [lane, wsp], acc0)
            plsc.store_scatter(out_v.at[b], [rows_b, wsp], acc1)

    pos_copy(0, 0).start()

    @pl.loop(0, NWIN, step=2)
    def _window(c):
        for b in range(2):
            ki = c + b
            nb = 1 - b
            pos_copy(ki, b).wait()

            @pl.when(ki + 1 < NWIN)
            def _():
                pos_copy(ki + 1, nb).start()

            compute_idx_wt(b)
            for cp in gather_copies(b):
                cp.start()

            @pl.when(ki >= 1)
            def _():
                @pl.when(ki >= 3)
                def _():
                    for cp in out_copies(ki - 3, nb):
                        cp.wait()
                for cp in gather_copies(nb):
                    cp.wait()
                blend(nb)
                for cp in out_copies(ki - 1, nb):
                    cp.start()

    for cp in out_copies(NWIN - 3, 1):
        cp.wait()
    for cp in gather_copies(1):
        cp.wait()
    blend(1)
    for cp in out_copies(NWIN - 1, 1):
        cp.start()
    for cp in out_copies(NWIN - 2, 0):
        cp.wait()
    for cp in out_copies(NWIN - 1, 1):
        cp.wait()


def _make_cp():
    cp = pltpu.CompilerParams()
    for field, val in (("needs_layout_passes", False),
                       ("use_tc_tiling_on_sc", False)):
        if field in pltpu.CompilerParams.__dataclass_fields__:
            cp = dataclasses.replace(cp, **{field: val})
    return cp


def kernel(positions, voxel_grid):
    mesh = plsc.VectorSubcoreMesh(core_axis_name="c", subcore_axis_name="s")
    cp = _make_cp()

    repack = pl.kernel(
        _repack_body,
        out_type=jax.ShapeDtypeStruct((V, DP), jnp.float32),
        mesh=mesh,
        compiler_params=cp,
        scratch_types=[
            pltpu.VMEM((2, 4, 32, 128), jnp.float32),
            pltpu.VMEM((2, RC, DP), jnp.float32),
            pltpu.SemaphoreType.DMA,
            pltpu.SemaphoreType.DMA,
        ],
    )

    main = pl.kernel(
        _main_body,
        out_type=jax.ShapeDtypeStruct((4 * NT * 8, 128), jnp.float32),
        mesh=mesh,
        compiler_params=cp,
        scratch_types=[
            pltpu.VMEM((2, 8, W), jnp.float32),
            pltpu.VMEM((2, 8, W), jnp.int32),
            pltpu.VMEM((2, 8, W), jnp.float32),
            pltpu.VMEM((2, 8, W, DP), jnp.float32),
            pltpu.VMEM((2, DP, W), jnp.float32),
            pltpu.SemaphoreType.DMA,
            pltpu.SemaphoreType.DMA,
            pltpu.SemaphoreType.DMA,
        ],
    )

    t_e = (jnp.pad(voxel_grid.T, ((0, 4), (0, 0)))
           .reshape(4, 8, VT, 128).transpose(0, 2, 1, 3).reshape(-1, 128))
    p_e = (jnp.pad(positions.T, ((0, 5), (0, 0)))
           .reshape(1, 8, NT, 128).transpose(0, 2, 1, 3).reshape(-1, 128))

    t32 = repack(t_e)
    o_e = main(p_e, t32)
    return (o_e.reshape(4, NT, 8, 128).transpose(0, 2, 1, 3)
            .reshape(DP, N)[:D].T)
